# baseline (speedup 1.0000x reference)
import functools

import jax
import jax.numpy as jnp
from jax import lax
from jax.experimental import pallas as pl
from jax.experimental.pallas import tpu as pltpu
from jax.experimental.pallas import tpu_sc as plsc

NUM_USERS = 5000
NUM_ITEMS = 5000
N = NUM_USERS + NUM_ITEMS
D = 128
LAYERS = 3

NC = 2
NS = 16
LANES = 16
DC = D // NC
VPR = DC // LANES
CHUNK = 128
GRP = 32
NBUF = 8
CPT = 160
NGRP = CPT // GRP
PIPE = GRP // NBUF
ROWS_PER_TILE = N // NS
SUB = 125
NSUB = ROWS_PER_TILE // SUB


def _sc_body(emb_flat, src_all, dst_e, val_e, out_e, tab1, tab2,
             r0b, r1b, r2b, r3b, r4b, r5b, r6b, r7b, src_g, dst_g, val_g,
             g0, g1, g2, g3, g4, g5, g6, g7,
             s0, s1, s2, s3, s4, s5, s6, s7, tab):
    rows = (r0b, r1b, r2b, r3b, r4b, r5b, r6b, r7b)
    gsem = (g0, g1, g2, g3, g4, g5, g6, g7)
    ssem = (s0, s1, s2, s3, s4, s5, s6, s7)
    c = lax.axis_index("c")
    s = lax.axis_index("s")
    r0 = pl.multiple_of(s * ROWS_PER_TILE, ROWS_PER_TILE)
    hb0 = c * N + r0
    cbase = pl.multiple_of(s * CPT, CPT)

    zero16 = jnp.zeros((LANES,), jnp.float32)
    stage = rows[0].at[pl.ds(0, SUB)]
    stage2 = rows[1].at[pl.ds(0, SUB)]

    def _zero_stage():
        def body(r, carry):
            for j in range(VPR):
                rows[0][r, pl.ds(j * LANES, LANES)] = zero16
            return carry
        lax.fori_loop(0, SUB, body, 0)

    _zero_stage()
    for m in range(NSUB):
        pltpu.sync_copy(stage, tab.at[pl.ds(r0 + m * SUB, SUB)])
    plsc.subcore_barrier()

    def scale_chunk(b, ci):
        @plsc.parallel_loop(0, CHUNK // LANES, unroll=2)
        def g_body(g):
            vv = val_g[ci, pl.ds(g * LANES, LANES)]
            for i in range(LANES):
                e = g * LANES + i
                v = vv[i]
                for j in range(VPR):
                    sl = pl.ds(j * LANES, LANES)
                    rows[b][e, sl] = rows[b][e, sl] * v

    for layer in range(LAYERS):
        tab_in = (emb_flat, tab1, tab2)[layer]
        tab_next = (tab1, tab2, None)[layer]

        def grp_body(grp, carry, tab_in=tab_in):
            gb = pl.multiple_of(cbase + grp * GRP, GRP)
            pltpu.sync_copy(src_all.at[c, pl.ds(gb, GRP)], src_g)
            pltpu.sync_copy(dst_e.at[pl.ds(gb, GRP)], dst_g)
            pltpu.sync_copy(val_e.at[pl.ds(gb, GRP)], val_g)

            def pipe_body(p, pcarry):
                ci0 = p * NBUF
                gathers = []
                for b in range(NBUF):
                    @pl.when(p > 0)
                    def _(b=b):
                        pltpu.make_async_copy(
                            rows[b], tab.at[dst_g.at[0]], ssem[b]).wait()
                    gathers.append(pltpu.async_copy(
                        tab_in.at[src_g.at[ci0 + b]], rows[b], gsem[b]))
                for b in range(NBUF):
                    gathers[b].wait()
                    scale_chunk(b, ci0 + b)
                    pltpu.async_copy(
                        rows[b], tab.at[dst_g.at[ci0 + b]], ssem[b], add=True)
                return pcarry

            lax.fori_loop(0, PIPE, pipe_body, 0)
            for b in range(NBUF):
                pltpu.make_async_copy(
                    rows[b], tab.at[dst_g.at[0]], ssem[b]).wait()
            return carry

        lax.fori_loop(0, NGRP, grp_body, 0)
        plsc.subcore_barrier()

        third = jnp.float32(1.0 / LAYERS)
        for m in range(NSUB):
            sl_rows = pl.ds(r0 + m * SUB, SUB)
            pltpu.sync_copy(tab.at[sl_rows], stage)
            if tab_next is not None:
                pltpu.sync_copy(
                    stage, tab_next.at[pl.ds(hb0 + m * SUB, SUB)])
            if layer > 0:
                pltpu.sync_copy(out_e.at[c, sl_rows], stage2)

            def merge_body(r, carry, layer=layer):
                for j in range(VPR):
                    sl = pl.ds(j * LANES, LANES)
                    x = rows[0][r, sl]
                    if layer > 0:
                        x = x + rows[1][r, sl]
                    if layer == LAYERS - 1:
                        x = x * third
                    rows[0][r, sl] = x
                return carry

            lax.fori_loop(0, SUB, merge_body, 0)
            pltpu.sync_copy(stage, out_e.at[c, sl_rows])

        if layer + 1 < LAYERS:
            _zero_stage()
            for m in range(NSUB):
                pltpu.sync_copy(stage, tab.at[pl.ds(r0 + m * SUB, SUB)])
        plsc.subcore_barrier()


@functools.partial(
    pl.kernel,
    out_type=(
        jax.ShapeDtypeStruct((NC, N, DC), jnp.float32),
        jax.ShapeDtypeStruct((NC * N, DC), jnp.float32),
        jax.ShapeDtypeStruct((NC * N, DC), jnp.float32),
    ),
    mesh=plsc.VectorSubcoreMesh(core_axis_name="c", subcore_axis_name="s"),
    compiler_params=pltpu.CompilerParams(use_tc_tiling_on_sc=False),
    scratch_types=(
        [pltpu.VMEM((CHUNK, DC), jnp.float32)] * 8
        + [
            pltpu.VMEM((GRP, CHUNK), jnp.int32),
            pltpu.VMEM((GRP, CHUNK), jnp.int32),
            pltpu.VMEM((GRP, CHUNK), jnp.float32),
        ]
        + [pltpu.SemaphoreType.DMA] * 16
        + [pltpu.VMEM_SHARED((N, DC), jnp.float32)]
    ),
)
def _propagate(emb_flat, src_all, dst_e, val_e, out_e, tab1, tab2, *scratch):
    _sc_body(emb_flat, src_all, dst_e, val_e, out_e, tab1, tab2, *scratch)


def kernel(perturbed, all_users, all_items, graph_indices, graph_values):
    del perturbed
    e = graph_values.shape[0]
    e_pad = NS * CPT * CHUNK
    pad = e_pad - e
    dst = jnp.pad(graph_indices[0], (0, pad)).reshape(-1, CHUNK)
    src = jnp.pad(graph_indices[1], (0, pad)).reshape(-1, CHUNK)
    src_all = jnp.stack([src, src + N])
    val = jnp.pad(graph_values, (0, pad)).reshape(-1, CHUNK)
    emb = jnp.concatenate([all_users, all_items], axis=0)
    emb_flat = emb.reshape(N, NC, DC).transpose(1, 0, 2).reshape(NC * N, DC)
    out, _, _ = _propagate(emb_flat, src_all, dst, val)
    emb_out = out.transpose(1, 0, 2).reshape(N, D)
    return (emb_out[:NUM_USERS], emb_out[NUM_USERS:])


# speedup vs baseline: 1.4781x; 1.4781x over previous
import functools

import jax
import jax.numpy as jnp
from jax import lax
from jax.experimental import pallas as pl
from jax.experimental.pallas import tpu as pltpu
from jax.experimental.pallas import tpu_sc as plsc

NUM_USERS = 5000
NUM_ITEMS = 5000
N = NUM_USERS + NUM_ITEMS
D = 128
LAYERS = 3

NC = 2
NS = 16
LANES = 16
DC = D // NC
VPR = DC // LANES
CHUNK = 128
GRP = 32
NBUF = 4
CPT = 160
NGRP = CPT // GRP
PIPE = GRP // NBUF
ROWS_PER_TILE = N // NS
SUB = 125
NSUB = ROWS_PER_TILE // SUB


def _sc_body(emb_in, src_e, dst_e, val_e, out_e,
             r0b, r1b, r2b, r3b, src_g, dst_g, val_g,
             g0, g1, g2, g3, s0, s1, s2, s3, isem, tabA, tabB):
    rows = (r0b, r1b, r2b, r3b)
    gsem = (g0, g1, g2, g3)
    ssem = (s0, s1, s2, s3)
    c = lax.axis_index("c")
    s = lax.axis_index("s")
    r0 = pl.multiple_of(s * ROWS_PER_TILE, ROWS_PER_TILE)
    cbase = pl.multiple_of(s * CPT, CPT)

    zero16 = jnp.zeros((LANES,), jnp.float32)
    stage = rows[0].at[pl.ds(0, SUB)]
    stage2 = rows[1].at[pl.ds(0, SUB)]

    def _zero_stage():
        def body(r, carry):
            for j in range(VPR):
                rows[0][r, pl.ds(j * LANES, LANES)] = zero16
            return carry
        lax.fori_loop(0, SUB, body, 0)

    for m in range(NSUB):
        pltpu.sync_copy(emb_in.at[c, pl.ds(r0 + m * SUB, SUB)], stage)
        pltpu.sync_copy(stage, tabA.at[pl.ds(r0 + m * SUB, SUB)])
    _zero_stage()
    for m in range(NSUB):
        pltpu.sync_copy(stage, tabB.at[pl.ds(r0 + m * SUB, SUB)])
    plsc.subcore_barrier()

    def scale_chunk(b, ci):
        @plsc.parallel_loop(0, CHUNK // LANES, unroll=2)
        def g_body(g):
            vv = val_g[ci, pl.ds(g * LANES, LANES)]
            for i in range(LANES):
                e = g * LANES + i
                v = vv[i]
                for j in range(VPR):
                    sl = pl.ds(j * LANES, LANES)
                    rows[b][e, sl] = rows[b][e, sl] * v

    for layer in range(LAYERS):
        tab_in = tabA if layer % 2 == 0 else tabB
        tab_out = tabB if layer % 2 == 0 else tabA

        def grp_body(grp, carry, tab_in=tab_in, tab_out=tab_out):
            gb = pl.multiple_of(cbase + grp * GRP, GRP)
            loads = [
                pltpu.async_copy(src_e.at[pl.ds(gb, GRP)], src_g, isem),
                pltpu.async_copy(dst_e.at[pl.ds(gb, GRP)], dst_g, isem),
                pltpu.async_copy(val_e.at[pl.ds(gb, GRP)], val_g, isem),
            ]
            for ld in loads:
                ld.wait()

            def pipe_body(p, pcarry):
                ci0 = p * NBUF
                gathers = []
                for b in range(NBUF):
                    @pl.when(p > 0)
                    def _(b=b):
                        pltpu.make_async_copy(
                            rows[b], tab_out.at[dst_g.at[0]], ssem[b]).wait()
                    gathers.append(pltpu.async_copy(
                        tab_in.at[src_g.at[ci0 + b]], rows[b], gsem[b]))
                for b in range(NBUF):
                    gathers[b].wait()
                    scale_chunk(b, ci0 + b)
                    pltpu.async_copy(
                        rows[b], tab_out.at[dst_g.at[ci0 + b]], ssem[b],
                        add=True)
                return pcarry

            lax.fori_loop(0, PIPE, pipe_body, 0)
            for b in range(NBUF):
                pltpu.make_async_copy(
                    rows[b], tab_out.at[dst_g.at[0]], ssem[b]).wait()
            return carry

        lax.fori_loop(0, NGRP, grp_body, 0)
        plsc.subcore_barrier()

        third = jnp.float32(1.0 / LAYERS)
        for m in range(NSUB):
            sl_rows = pl.ds(r0 + m * SUB, SUB)
            pltpu.sync_copy(tab_out.at[sl_rows], stage)
            if layer > 0:
                pltpu.sync_copy(out_e.at[c, sl_rows], stage2)

            def merge_body(r, carry, layer=layer):
                for j in range(VPR):
                    sl = pl.ds(j * LANES, LANES)
                    x = rows[0][r, sl]
                    if layer > 0:
                        x = x + rows[1][r, sl]
                    if layer == LAYERS - 1:
                        x = x * third
                    rows[0][r, sl] = x
                return carry

            lax.fori_loop(0, SUB, merge_body, 0)
            pltpu.sync_copy(stage, out_e.at[c, sl_rows])

        if layer + 1 < LAYERS:
            _zero_stage()
            for m in range(NSUB):
                pltpu.sync_copy(stage, tab_in.at[pl.ds(r0 + m * SUB, SUB)])
        plsc.subcore_barrier()


@functools.partial(
    pl.kernel,
    out_type=jax.ShapeDtypeStruct((NC, N, DC), jnp.float32),
    mesh=plsc.VectorSubcoreMesh(core_axis_name="c", subcore_axis_name="s"),
    compiler_params=pltpu.CompilerParams(use_tc_tiling_on_sc=False),
    scratch_types=[
        pltpu.VMEM((CHUNK, DC), jnp.float32),
        pltpu.VMEM((CHUNK, DC), jnp.float32),
        pltpu.VMEM((CHUNK, DC), jnp.float32),
        pltpu.VMEM((CHUNK, DC), jnp.float32),
        pltpu.VMEM((GRP, CHUNK), jnp.int32),
        pltpu.VMEM((GRP, CHUNK), jnp.int32),
        pltpu.VMEM((GRP, CHUNK), jnp.float32),
        pltpu.SemaphoreType.DMA,
        pltpu.SemaphoreType.DMA,
        pltpu.SemaphoreType.DMA,
        pltpu.SemaphoreType.DMA,
        pltpu.SemaphoreType.DMA,
        pltpu.SemaphoreType.DMA,
        pltpu.SemaphoreType.DMA,
        pltpu.SemaphoreType.DMA,
        pltpu.SemaphoreType.DMA,
        pltpu.VMEM_SHARED((N, DC), jnp.float32),
        pltpu.VMEM_SHARED((N, DC), jnp.float32),
    ],
)
def _propagate(emb_in, src_e, dst_e, val_e, out_e, *scratch):
    _sc_body(emb_in, src_e, dst_e, val_e, out_e, *scratch)


def kernel(perturbed, all_users, all_items, graph_indices, graph_values):
    del perturbed
    e = graph_values.shape[0]
    e_pad = NS * CPT * CHUNK
    pad = e_pad - e
    dst = jnp.pad(graph_indices[0], (0, pad)).reshape(-1, CHUNK)
    src = jnp.pad(graph_indices[1], (0, pad)).reshape(-1, CHUNK)
    val = jnp.pad(graph_values, (0, pad)).reshape(-1, CHUNK)
    emb = jnp.concatenate([all_users, all_items], axis=0)
    emb_in = emb.reshape(N, NC, DC).transpose(1, 0, 2)
    out = _propagate(emb_in, src, dst, val)
    emb_out = out.transpose(1, 0, 2).reshape(N, D)
    return (emb_out[:NUM_USERS], emb_out[NUM_USERS:])


# speedup vs baseline: 1.4864x; 1.0056x over previous
import functools

import jax
import jax.numpy as jnp
from jax import lax
from jax.experimental import pallas as pl
from jax.experimental.pallas import tpu as pltpu
from jax.experimental.pallas import tpu_sc as plsc

NUM_USERS = 5000
NUM_ITEMS = 5000
N = NUM_USERS + NUM_ITEMS
D = 128
LAYERS = 3

NC = 2
NS = 16
LANES = 16
DC = D // NC
VPR = DC // LANES
CHUNK = 128
GRP = 40
NBUF = 4
CPT = 160
NGRP = CPT // GRP
PIPE = GRP // NBUF
ROWS_PER_TILE = N // NS
SUB = 125
NSUB = ROWS_PER_TILE // SUB


def _sc_body(emb_in, src_e, dst_e, val_e, out_e,
             r0b, r1b, r2b, r3b, src_g, dst_g, val_g,
             g0, g1, g2, g3, s0, s1, s2, s3, isem, tabA, tabB):
    rows = (r0b, r1b, r2b, r3b)
    gsem = (g0, g1, g2, g3)
    ssem = (s0, s1, s2, s3)
    c = lax.axis_index("c")
    s = lax.axis_index("s")
    r0 = pl.multiple_of(s * ROWS_PER_TILE, ROWS_PER_TILE)
    cbase = pl.multiple_of(s * CPT, CPT)

    zero16 = jnp.zeros((LANES,), jnp.float32)
    stage = rows[0].at[pl.ds(0, SUB)]
    stage2 = rows[1].at[pl.ds(0, SUB)]

    def _zero_stage():
        def body(r, carry):
            for j in range(VPR):
                rows[0][r, pl.ds(j * LANES, LANES)] = zero16
            return carry
        lax.fori_loop(0, SUB, body, 0)

    for m in range(NSUB):
        pltpu.sync_copy(emb_in.at[c, pl.ds(r0 + m * SUB, SUB)], stage)
        pltpu.sync_copy(stage, tabA.at[pl.ds(r0 + m * SUB, SUB)])
    _zero_stage()
    for m in range(NSUB):
        pltpu.sync_copy(stage, tabB.at[pl.ds(r0 + m * SUB, SUB)])
    plsc.subcore_barrier()

    def scale_chunk(b, ci):
        @plsc.parallel_loop(0, CHUNK // LANES, unroll=2)
        def g_body(g):
            vv = val_g[ci, pl.ds(g * LANES, LANES)]
            for i in range(LANES):
                e = g * LANES + i
                v = vv[i]
                for j in range(VPR):
                    sl = pl.ds(j * LANES, LANES)
                    rows[b][e, sl] = rows[b][e, sl] * v

    for layer in range(LAYERS):
        tab_in = tabA if layer % 2 == 0 else tabB
        tab_out = tabB if layer % 2 == 0 else tabA

        def grp_body(grp, carry, tab_in=tab_in, tab_out=tab_out):
            gb = pl.multiple_of(cbase + grp * GRP, GRP)
            loads = [
                pltpu.async_copy(src_e.at[pl.ds(gb, GRP)], src_g, isem),
                pltpu.async_copy(dst_e.at[pl.ds(gb, GRP)], dst_g, isem),
                pltpu.async_copy(val_e.at[pl.ds(gb, GRP)], val_g, isem),
            ]
            for ld in loads:
                ld.wait()

            def pipe_body(p, pcarry):
                ci0 = p * NBUF
                gathers = []
                for b in range(NBUF):
                    @pl.when(p > 0)
                    def _(b=b):
                        pltpu.make_async_copy(
                            rows[b], tab_out.at[dst_g.at[0]], ssem[b]).wait()
                    gathers.append(pltpu.async_copy(
                        tab_in.at[src_g.at[ci0 + b]], rows[b], gsem[b]))
                for b in range(NBUF):
                    gathers[b].wait()
                    scale_chunk(b, ci0 + b)
                    pltpu.async_copy(
                        rows[b], tab_out.at[dst_g.at[ci0 + b]], ssem[b],
                        add=True)
                return pcarry

            lax.fori_loop(0, PIPE, pipe_body, 0)
            for b in range(NBUF):
                pltpu.make_async_copy(
                    rows[b], tab_out.at[dst_g.at[0]], ssem[b]).wait()
            return carry

        lax.fori_loop(0, NGRP, grp_body, 0)
        plsc.subcore_barrier()

        third = jnp.float32(1.0 / LAYERS)
        for m in range(NSUB):
            sl_rows = pl.ds(r0 + m * SUB, SUB)
            pltpu.sync_copy(tab_out.at[sl_rows], stage)
            if layer > 0:
                pltpu.sync_copy(out_e.at[c, sl_rows], stage2)

            def merge_body(r, carry, layer=layer):
                for j in range(VPR):
                    sl = pl.ds(j * LANES, LANES)
                    x = rows[0][r, sl]
                    if layer > 0:
                        x = x + rows[1][r, sl]
                    if layer == LAYERS - 1:
                        x = x * third
                    rows[0][r, sl] = x
                return carry

            lax.fori_loop(0, SUB, merge_body, 0)
            pltpu.sync_copy(stage, out_e.at[c, sl_rows])

        if layer + 1 < LAYERS:
            _zero_stage()
            for m in range(NSUB):
                pltpu.sync_copy(stage, tab_in.at[pl.ds(r0 + m * SUB, SUB)])
        plsc.subcore_barrier()


@functools.partial(
    pl.kernel,
    out_type=jax.ShapeDtypeStruct((NC, N, DC), jnp.float32),
    mesh=plsc.VectorSubcoreMesh(core_axis_name="c", subcore_axis_name="s"),
    compiler_params=pltpu.CompilerParams(use_tc_tiling_on_sc=False),
    scratch_types=[
        pltpu.VMEM((CHUNK, DC), jnp.float32),
        pltpu.VMEM((CHUNK, DC), jnp.float32),
        pltpu.VMEM((CHUNK, DC), jnp.float32),
        pltpu.VMEM((CHUNK, DC), jnp.float32),
        pltpu.VMEM((GRP, CHUNK), jnp.int32),
        pltpu.VMEM((GRP, CHUNK), jnp.int32),
        pltpu.VMEM((GRP, CHUNK), jnp.float32),
        pltpu.SemaphoreType.DMA,
        pltpu.SemaphoreType.DMA,
        pltpu.SemaphoreType.DMA,
        pltpu.SemaphoreType.DMA,
        pltpu.SemaphoreType.DMA,
        pltpu.SemaphoreType.DMA,
        pltpu.SemaphoreType.DMA,
        pltpu.SemaphoreType.DMA,
        pltpu.SemaphoreType.DMA,
        pltpu.VMEM_SHARED((N, DC), jnp.float32),
        pltpu.VMEM_SHARED((N, DC), jnp.float32),
    ],
)
def _propagate(emb_in, src_e, dst_e, val_e, out_e, *scratch):
    _sc_body(emb_in, src_e, dst_e, val_e, out_e, *scratch)


def kernel(perturbed, all_users, all_items, graph_indices, graph_values):
    del perturbed
    e = graph_values.shape[0]
    e_pad = NS * CPT * CHUNK
    pad = e_pad - e
    dst = jnp.pad(graph_indices[0], (0, pad)).reshape(-1, CHUNK)
    src = jnp.pad(graph_indices[1], (0, pad)).reshape(-1, CHUNK)
    val = jnp.pad(graph_values, (0, pad)).reshape(-1, CHUNK)
    emb = jnp.concatenate([all_users, all_items], axis=0)
    emb_in = emb.reshape(N, NC, DC).transpose(1, 0, 2)
    out = _propagate(emb_in, src, dst, val)
    emb_out = out.transpose(1, 0, 2).reshape(N, D)
    return (emb_out[:NUM_USERS], emb_out[NUM_USERS:])
